# baseline (speedup 1.0000x reference)
import functools

import jax
import jax.numpy as jnp
from jax import lax
from jax.experimental import pallas as pl
from jax.experimental.pallas import tpu as pltpu
from jax.experimental.pallas import tpu_sc as plsc

N = 10000
E = 320000
IN_DIM = 128
HID = 64
NUM_COLORS = 16

NC = 2
NS = 16
CHUNK = 128
NQ = E // CHUNK
NW = NC * NS
N_CHUNKS = 79
N_PAD = 10240
ROWS_PER_TILE = N_PAD // NS

_mesh = plsc.VectorSubcoreMesh(
    core_axis_name="c", subcore_axis_name="s", num_cores=NC, num_subcores=NS)
_sc_params = pltpu.CompilerParams(use_tc_tiling_on_sc=False)


@functools.partial(
    pl.kernel,
    out_type=jax.ShapeDtypeStruct((NC, N_PAD), jnp.float32),
    mesh=_mesh,
    scratch_types=[
        pltpu.VMEM((N_CHUNKS, CHUNK), jnp.int32),
        pltpu.VMEM((CHUNK,), jnp.float32),
        pltpu.VMEM_SHARED((N_PAD,), jnp.float32),
        pltpu.SemaphoreType.DMA,
    ],
    compiler_params=_sc_params,
)
def _sc_degree(dst_hbm, ones_hbm, zeros_hbm, out_hbm, dst_v, ones_v, deg_sh, sem):
    c = lax.axis_index("c")
    s = lax.axis_index("s")
    wid = c * NS + s
    q_lo = (wid * NQ) // NW
    nq = ((wid + 1) * NQ) // NW - q_lo
    base = s * ROWS_PER_TILE
    pltpu.sync_copy(zeros_hbm, deg_sh.at[pl.ds(base, ROWS_PER_TILE)])
    pltpu.sync_copy(ones_hbm, ones_v)
    pltpu.sync_copy(dst_hbm.at[pl.ds(q_lo, N_CHUNKS)], dst_v)
    plsc.subcore_barrier()

    def body(j, carry):
        pltpu.sync_copy(ones_v, deg_sh.at[dst_v.at[j]], add=True)
        return carry

    lax.fori_loop(0, nq, body, 0)
    plsc.subcore_barrier()
    pltpu.sync_copy(deg_sh.at[pl.ds(base, ROWS_PER_TILE)],
                    out_hbm.at[c, pl.ds(base, ROWS_PER_TILE)])


@functools.partial(
    pl.kernel,
    out_type=jax.ShapeDtypeStruct((NC, N_PAD, HID), jnp.float32),
    mesh=_mesh,
    scratch_types=[
        pltpu.VMEM((N_CHUNKS, CHUNK), jnp.int32),
        pltpu.VMEM((N_CHUNKS, CHUNK), jnp.int32),
        pltpu.VMEM((4, CHUNK, HID), jnp.float32),
        pltpu.VMEM_SHARED((N_PAD, HID), jnp.float32),
        pltpu.SemaphoreType.DMA((4,)),
        pltpu.SemaphoreType.DMA((4,)),
    ],
    compiler_params=_sc_params,
)
def _sc_aggregate(src_hbm, dst_hbm, zeros_hbm, tab_hbm, out_hbm,
                  src_v, dst_v, rows_v, acc_sh, sem_g, sem_s):
    c = lax.axis_index("c")
    s = lax.axis_index("s")
    wid = c * NS + s
    q_lo = (wid * NQ) // NW
    nq = ((wid + 1) * NQ) // NW - q_lo
    base = s * ROWS_PER_TILE
    pltpu.sync_copy(zeros_hbm, acc_sh.at[pl.ds(base, ROWS_PER_TILE)])
    pltpu.sync_copy(src_hbm.at[pl.ds(q_lo, N_CHUNKS)], src_v)
    pltpu.sync_copy(dst_hbm.at[pl.ds(q_lo, N_CHUNKS)], dst_v)
    plsc.subcore_barrier()

    NBUF = 4
    for j0 in range(NBUF - 1):
        pltpu.async_copy(tab_hbm.at[src_v.at[j0]], rows_v.at[j0], sem_g.at[j0])

    def body(j, carry):
        b = lax.rem(j, NBUF)
        pltpu.make_async_copy(tab_hbm.at[src_v.at[j]], rows_v.at[b],
                              sem_g.at[b]).wait()
        pltpu.async_copy(rows_v.at[b], acc_sh.at[dst_v.at[j]], sem_s.at[b],
                         add=True)
        k = j + NBUF - 1

        @pl.when(k < nq)
        def _():
            bk = lax.rem(k, NBUF)

            @pl.when(j >= 1)
            def _():
                pltpu.make_async_copy(rows_v.at[bk], acc_sh.at[dst_v.at[j - 1]],
                                      sem_s.at[bk]).wait()

            pltpu.async_copy(tab_hbm.at[src_v.at[k]], rows_v.at[bk], sem_g.at[bk])

        return carry

    lax.fori_loop(0, nq, body, 0)
    for i in range(NBUF):
        jj = nq - NBUF + i
        pltpu.make_async_copy(rows_v.at[jj % NBUF], acc_sh.at[dst_v.at[jj]],
                              sem_s.at[jj % NBUF]).wait()
    plsc.subcore_barrier()
    pltpu.sync_copy(acc_sh.at[pl.ds(base, ROWS_PER_TILE)],
                    out_hbm.at[c, pl.ds(base, ROWS_PER_TILE)])


GRID = 10
BLK = N_PAD // GRID


def _blk(shape2):
    return pl.BlockSpec((BLK,) + shape2, lambda i: (i,) + (0,) * len(shape2))


def _full(shape):
    return pl.BlockSpec(shape, lambda i: (0,) * len(shape))


def _tc1a_body(x_ref, w_ref, h_ref):
    h_ref[...] = jnp.dot(x_ref[...], w_ref[...],
                         preferred_element_type=jnp.float32)


def _tc1b_body(deg_ref, h_ref, p_ref, dinv_ref):
    d = deg_ref[:, 0:1] + deg_ref[:, 1:2] + 1.0
    dinv = lax.rsqrt(d)
    p_ref[...] = h_ref[...] * dinv
    dinv_ref[...] = dinv


def _tc2_body(acc_ref, p_ref, dinv_ref, b_ref, w_ref, out_ref):
    agg = acc_ref[0] + acc_ref[1] + p_ref[...]
    h = jnp.maximum(agg * dinv_ref[...] + b_ref[...], 0.0)
    out_ref[...] = jnp.dot(h, w_ref[...],
                           preferred_element_type=jnp.float32) * dinv_ref[...]


def _tc3_body(acc_ref, p_ref, dinv_ref, b_ref, w_ref, bh_ref, mask_ref, out_ref):
    agg = acc_ref[0] + acc_ref[1] + p_ref[...]
    h = jnp.maximum(agg * dinv_ref[...] + b_ref[...], 0.0)
    logits = jnp.dot(h, w_ref[...], preferred_element_type=jnp.float32) + bh_ref[...]
    neg = jnp.float32(jnp.finfo(jnp.float32).min)
    out_ref[...] = jnp.where(mask_ref[...] > 0, logits, neg)


def kernel(x, edge_index, action_mask, W1, b1, W2, b2, Wh, bh):
    f32 = jnp.float32
    srcp = edge_index[0].reshape(NQ, CHUNK)
    dstp = edge_index[1].reshape(NQ, CHUNK)
    xpad = jnp.pad(x, ((0, N_PAD - N), (0, 0)))
    maskp = jnp.pad(action_mask, ((0, N_PAD - N), (0, 0))).astype(jnp.int32)
    zeros1 = jnp.zeros((ROWS_PER_TILE,), f32)
    zeros2 = jnp.zeros((ROWS_PER_TILE, HID), f32)
    ones_c = jnp.ones((CHUNK,), f32)
    b1r = b1.reshape(1, HID)
    b2r = b2.reshape(1, HID)
    bhr = bh.reshape(1, NUM_COLORS)

    deg = _sc_degree(dstp, ones_c, zeros1)
    degT = deg.T
    acc_spec = pl.BlockSpec((NC, BLK, HID), lambda i: (0, i, 0))
    h1 = pl.pallas_call(
        _tc1a_body,
        grid=(GRID,),
        in_specs=[_blk((IN_DIM,)), _full((IN_DIM, HID))],
        out_specs=_blk((HID,)),
        out_shape=jax.ShapeDtypeStruct((N_PAD, HID), f32),
    )(xpad, W1)

    p1s, dinv = pl.pallas_call(
        _tc1b_body,
        grid=(GRID,),
        in_specs=[_blk((2,)), _blk((HID,))],
        out_specs=[_blk((HID,)), _blk((1,))],
        out_shape=[jax.ShapeDtypeStruct((N_PAD, HID), f32),
                   jax.ShapeDtypeStruct((N_PAD, 1), f32)],
    )(degT, h1)
    acc1 = _sc_aggregate(srcp, dstp, zeros2, p1s)

    p2s = pl.pallas_call(
        _tc2_body,
        grid=(GRID,),
        in_specs=[acc_spec, _blk((HID,)), _blk((1,)), _full((1, HID)),
                  _full((HID, HID))],
        out_specs=_blk((HID,)),
        out_shape=jax.ShapeDtypeStruct((N_PAD, HID), f32),
    )(acc1, p1s, dinv, b1r, W2)
    acc2 = _sc_aggregate(srcp, dstp, zeros2, p2s)

    outp = pl.pallas_call(
        _tc3_body,
        grid=(GRID,),
        in_specs=[acc_spec, _blk((HID,)), _blk((1,)), _full((1, HID)),
                  _full((HID, NUM_COLORS)), _full((1, NUM_COLORS)),
                  _blk((NUM_COLORS,))],
        out_specs=_blk((NUM_COLORS,)),
        out_shape=jax.ShapeDtypeStruct((N_PAD, NUM_COLORS), f32),
    )(acc2, p2s, dinv, b2r, Wh, bhr, maskp)
    return outp[:N].reshape(-1)


# speedup vs baseline: 1.0311x; 1.0311x over previous
import functools

import jax
import jax.numpy as jnp
from jax import lax
from jax.experimental import pallas as pl
from jax.experimental.pallas import tpu as pltpu
from jax.experimental.pallas import tpu_sc as plsc

N = 10000
E = 320000
IN_DIM = 128
HID = 64
NUM_COLORS = 16

NC = 2
NS = 16
CHUNK = 128
NQ = E // CHUNK
NW = NC * NS
N_CHUNKS = 79
N_PAD = 10240
ROWS_PER_TILE = N_PAD // NS

_mesh = plsc.VectorSubcoreMesh(
    core_axis_name="c", subcore_axis_name="s", num_cores=NC, num_subcores=NS)
_sc_params = pltpu.CompilerParams(use_tc_tiling_on_sc=False)


@functools.partial(
    pl.kernel,
    out_type=jax.ShapeDtypeStruct((NC, N_PAD), jnp.float32),
    mesh=_mesh,
    scratch_types=[
        pltpu.VMEM((N_CHUNKS, CHUNK), jnp.int32),
        pltpu.VMEM((CHUNK,), jnp.float32),
        pltpu.VMEM_SHARED((N_PAD,), jnp.float32),
        pltpu.SemaphoreType.DMA,
    ],
    compiler_params=_sc_params,
)
def _sc_degree(dst_hbm, ones_hbm, zeros_hbm, out_hbm, dst_v, ones_v, deg_sh, sem):
    c = lax.axis_index("c")
    s = lax.axis_index("s")
    wid = c * NS + s
    q_lo = (wid * NQ) // NW
    nq = ((wid + 1) * NQ) // NW - q_lo
    base = s * ROWS_PER_TILE
    pltpu.sync_copy(zeros_hbm, deg_sh.at[pl.ds(base, ROWS_PER_TILE)])
    pltpu.sync_copy(ones_hbm, ones_v)
    pltpu.sync_copy(dst_hbm.at[pl.ds(q_lo, N_CHUNKS)], dst_v)
    plsc.subcore_barrier()

    def body(j, carry):
        pltpu.sync_copy(ones_v, deg_sh.at[dst_v.at[j]], add=True)
        return carry

    lax.fori_loop(0, nq, body, 0)
    plsc.subcore_barrier()
    pltpu.sync_copy(deg_sh.at[pl.ds(base, ROWS_PER_TILE)],
                    out_hbm.at[c, pl.ds(base, ROWS_PER_TILE)])


@functools.partial(
    pl.kernel,
    out_type=jax.ShapeDtypeStruct((NC, N_PAD, HID), jnp.float32),
    mesh=_mesh,
    scratch_types=[
        pltpu.VMEM((N_CHUNKS, CHUNK), jnp.int32),
        pltpu.VMEM((N_CHUNKS, CHUNK), jnp.int32),
        pltpu.VMEM((4, CHUNK, HID), jnp.float32),
        pltpu.VMEM_SHARED((N_PAD, HID), jnp.float32),
        pltpu.SemaphoreType.DMA((4,)),
        pltpu.SemaphoreType.DMA((4,)),
    ],
    compiler_params=_sc_params,
)
def _sc_aggregate(src_hbm, dst_hbm, zeros_hbm, tab_hbm, out_hbm,
                  src_v, dst_v, rows_v, acc_sh, sem_g, sem_s):
    c = lax.axis_index("c")
    s = lax.axis_index("s")
    wid = c * NS + s
    q_lo = (wid * NQ) // NW
    nq = ((wid + 1) * NQ) // NW - q_lo
    base = s * ROWS_PER_TILE
    pltpu.sync_copy(zeros_hbm, acc_sh.at[pl.ds(base, ROWS_PER_TILE)])
    pltpu.sync_copy(src_hbm.at[pl.ds(q_lo, N_CHUNKS)], src_v)
    pltpu.sync_copy(dst_hbm.at[pl.ds(q_lo, N_CHUNKS)], dst_v)
    plsc.subcore_barrier()

    NBUF = 4
    for j0 in range(NBUF - 1):
        pltpu.async_copy(tab_hbm.at[src_v.at[j0]], rows_v.at[j0], sem_g.at[j0])

    def body(j, carry):
        b = lax.rem(j, NBUF)
        pltpu.make_async_copy(tab_hbm.at[src_v.at[j]], rows_v.at[b],
                              sem_g.at[b]).wait()
        pltpu.async_copy(rows_v.at[b], acc_sh.at[dst_v.at[j]], sem_s.at[b],
                         add=True)
        k = j + NBUF - 1

        @pl.when(k < nq)
        def _():
            bk = lax.rem(k, NBUF)

            @pl.when(j >= 1)
            def _():
                pltpu.make_async_copy(rows_v.at[bk], acc_sh.at[dst_v.at[j - 1]],
                                      sem_s.at[bk]).wait()

            pltpu.async_copy(tab_hbm.at[src_v.at[k]], rows_v.at[bk], sem_g.at[bk])

        return carry

    lax.fori_loop(0, nq, body, 0)
    for i in range(NBUF):
        jj = nq - NBUF + i
        pltpu.make_async_copy(rows_v.at[jj % NBUF], acc_sh.at[dst_v.at[jj]],
                              sem_s.at[jj % NBUF]).wait()
    plsc.subcore_barrier()
    pltpu.sync_copy(acc_sh.at[pl.ds(base, ROWS_PER_TILE)],
                    out_hbm.at[c, pl.ds(base, ROWS_PER_TILE)])


GRID = 4
BLK = N_PAD // GRID


def _blk(shape2):
    return pl.BlockSpec((BLK,) + shape2, lambda i: (i,) + (0,) * len(shape2))


def _full(shape):
    return pl.BlockSpec(shape, lambda i: (0,) * len(shape))


def _tc1a_body(x_ref, w_ref, h_ref):
    h_ref[...] = jnp.dot(x_ref[...], w_ref[...],
                         preferred_element_type=jnp.float32)


def _tc1b_body(deg_ref, h_ref, p_ref, dinv_ref):
    d = deg_ref[:, 0:1] + deg_ref[:, 1:2] + 1.0
    dinv = lax.rsqrt(d)
    p_ref[...] = h_ref[...] * dinv
    dinv_ref[...] = dinv


def _tc2_body(acc_ref, p_ref, dinv_ref, b_ref, w_ref, out_ref):
    agg = acc_ref[0] + acc_ref[1] + p_ref[...]
    h = jnp.maximum(agg * dinv_ref[...] + b_ref[...], 0.0)
    out_ref[...] = jnp.dot(h, w_ref[...],
                           preferred_element_type=jnp.float32) * dinv_ref[...]


def _tc3_body(acc_ref, p_ref, dinv_ref, b_ref, w_ref, bh_ref, mask_ref, out_ref):
    agg = acc_ref[0] + acc_ref[1] + p_ref[...]
    h = jnp.maximum(agg * dinv_ref[...] + b_ref[...], 0.0)
    logits = jnp.dot(h, w_ref[...], preferred_element_type=jnp.float32) + bh_ref[...]
    neg = jnp.float32(jnp.finfo(jnp.float32).min)
    out_ref[...] = jnp.where(mask_ref[...] > 0, logits, neg)


def kernel(x, edge_index, action_mask, W1, b1, W2, b2, Wh, bh):
    f32 = jnp.float32
    srcp = edge_index[0].reshape(NQ, CHUNK)
    dstp = edge_index[1].reshape(NQ, CHUNK)
    xpad = jnp.pad(x, ((0, N_PAD - N), (0, 0)))
    maskp = jnp.pad(action_mask, ((0, N_PAD - N), (0, 0))).astype(jnp.int32)
    zeros1 = jnp.zeros((ROWS_PER_TILE,), f32)
    zeros2 = jnp.zeros((ROWS_PER_TILE, HID), f32)
    ones_c = jnp.ones((CHUNK,), f32)
    b1r = b1.reshape(1, HID)
    b2r = b2.reshape(1, HID)
    bhr = bh.reshape(1, NUM_COLORS)

    deg = _sc_degree(dstp, ones_c, zeros1)
    degT = deg.T
    acc_spec = pl.BlockSpec((NC, BLK, HID), lambda i: (0, i, 0))
    h1 = pl.pallas_call(
        _tc1a_body,
        grid=(GRID,),
        in_specs=[_blk((IN_DIM,)), _full((IN_DIM, HID))],
        out_specs=_blk((HID,)),
        out_shape=jax.ShapeDtypeStruct((N_PAD, HID), f32),
    )(xpad, W1)

    p1s, dinv = pl.pallas_call(
        _tc1b_body,
        grid=(GRID,),
        in_specs=[_blk((2,)), _blk((HID,))],
        out_specs=[_blk((HID,)), _blk((1,))],
        out_shape=[jax.ShapeDtypeStruct((N_PAD, HID), f32),
                   jax.ShapeDtypeStruct((N_PAD, 1), f32)],
    )(degT, h1)
    acc1 = _sc_aggregate(srcp, dstp, zeros2, p1s)

    p2s = pl.pallas_call(
        _tc2_body,
        grid=(GRID,),
        in_specs=[acc_spec, _blk((HID,)), _blk((1,)), _full((1, HID)),
                  _full((HID, HID))],
        out_specs=_blk((HID,)),
        out_shape=jax.ShapeDtypeStruct((N_PAD, HID), f32),
    )(acc1, p1s, dinv, b1r, W2)
    acc2 = _sc_aggregate(srcp, dstp, zeros2, p2s)

    outp = pl.pallas_call(
        _tc3_body,
        grid=(GRID,),
        in_specs=[acc_spec, _blk((HID,)), _blk((1,)), _full((1, HID)),
                  _full((HID, NUM_COLORS)), _full((1, NUM_COLORS)),
                  _blk((NUM_COLORS,))],
        out_specs=_blk((NUM_COLORS,)),
        out_shape=jax.ShapeDtypeStruct((N_PAD, NUM_COLORS), f32),
    )(acc2, p2s, dinv, b2r, Wh, bhr, maskp)
    return outp[:N].reshape(-1)


# speedup vs baseline: 1.0828x; 1.0501x over previous
import functools

import jax
import jax.numpy as jnp
from jax import lax
from jax.experimental import pallas as pl
from jax.experimental.pallas import tpu as pltpu
from jax.experimental.pallas import tpu_sc as plsc

N = 10000
E = 320000
IN_DIM = 128
HID = 64
NUM_COLORS = 16

NC = 2
NS = 16
CHUNK = 128
NQ = E // CHUNK
NW = NC * NS
NC_EDGE = 2
N_CHUNKS = 79
N_PAD = 10240
ROWS_PER_TILE = N_PAD // NS

_mesh = plsc.VectorSubcoreMesh(
    core_axis_name="c", subcore_axis_name="s", num_cores=NC, num_subcores=NS)
_sc_params = pltpu.CompilerParams(use_tc_tiling_on_sc=False)


@functools.partial(
    pl.kernel,
    out_type=jax.ShapeDtypeStruct((NC, N_PAD), jnp.float32),
    mesh=_mesh,
    scratch_types=[
        pltpu.VMEM((N_CHUNKS, CHUNK), jnp.int32),
        pltpu.VMEM((CHUNK,), jnp.float32),
        pltpu.VMEM_SHARED((N_PAD,), jnp.float32),
        pltpu.SemaphoreType.DMA,
    ],
    compiler_params=_sc_params,
)
def _sc_degree(edges_hbm, ones_hbm, zeros_hbm, out_hbm, dst_v, ones_v, deg_sh, sem):
    c = lax.axis_index("c")
    s = lax.axis_index("s")
    wid = c * NS + s
    q_lo = (wid * NQ) // NW
    nq = ((wid + 1) * NQ) // NW - q_lo
    base = s * ROWS_PER_TILE
    pltpu.sync_copy(zeros_hbm, deg_sh.at[pl.ds(base, ROWS_PER_TILE)])
    pltpu.sync_copy(ones_hbm, ones_v)
    pltpu.sync_copy(edges_hbm.at[1, pl.ds(q_lo, N_CHUNKS)], dst_v)
    plsc.subcore_barrier()

    def body(j, carry):
        pltpu.sync_copy(ones_v, deg_sh.at[dst_v.at[j]], add=True)
        return carry

    lax.fori_loop(0, nq, body, 0)
    plsc.subcore_barrier()
    pltpu.sync_copy(deg_sh.at[pl.ds(base, ROWS_PER_TILE)],
                    out_hbm.at[c, pl.ds(base, ROWS_PER_TILE)])


@functools.partial(
    pl.kernel,
    out_type=jax.ShapeDtypeStruct((NC, N_PAD, HID), jnp.float32),
    mesh=_mesh,
    scratch_types=[
        pltpu.VMEM((N_CHUNKS, CHUNK), jnp.int32),
        pltpu.VMEM((N_CHUNKS, CHUNK), jnp.int32),
        pltpu.VMEM((4, CHUNK, HID), jnp.float32),
        pltpu.VMEM_SHARED((N_PAD, HID), jnp.float32),
        pltpu.SemaphoreType.DMA((4,)),
        pltpu.SemaphoreType.DMA((4,)),
    ],
    compiler_params=_sc_params,
)
def _sc_aggregate(edges_hbm, zeros_hbm, tab_hbm, out_hbm,
                  src_v, dst_v, rows_v, acc_sh, sem_g, sem_s):
    c = lax.axis_index("c")
    s = lax.axis_index("s")
    wid = c * NS + s
    q_lo = (wid * NQ) // NW
    nq = ((wid + 1) * NQ) // NW - q_lo
    base = s * ROWS_PER_TILE
    pltpu.sync_copy(zeros_hbm, acc_sh.at[pl.ds(base, ROWS_PER_TILE)])
    pltpu.sync_copy(edges_hbm.at[0, pl.ds(q_lo, N_CHUNKS)], src_v)
    pltpu.sync_copy(edges_hbm.at[1, pl.ds(q_lo, N_CHUNKS)], dst_v)
    plsc.subcore_barrier()

    NBUF = 4
    for j0 in range(NBUF - 1):
        pltpu.async_copy(tab_hbm.at[src_v.at[j0]], rows_v.at[j0], sem_g.at[j0])

    def body(j, carry):
        b = lax.rem(j, NBUF)
        pltpu.make_async_copy(tab_hbm.at[src_v.at[j]], rows_v.at[b],
                              sem_g.at[b]).wait()
        pltpu.async_copy(rows_v.at[b], acc_sh.at[dst_v.at[j]], sem_s.at[b],
                         add=True)
        k = j + NBUF - 1

        @pl.when(k < nq)
        def _():
            bk = lax.rem(k, NBUF)

            @pl.when(j >= 1)
            def _():
                pltpu.make_async_copy(rows_v.at[bk], acc_sh.at[dst_v.at[j - 1]],
                                      sem_s.at[bk]).wait()

            pltpu.async_copy(tab_hbm.at[src_v.at[k]], rows_v.at[bk], sem_g.at[bk])

        return carry

    lax.fori_loop(0, nq, body, 0)
    for i in range(NBUF):
        jj = nq - NBUF + i
        pltpu.make_async_copy(rows_v.at[jj % NBUF], acc_sh.at[dst_v.at[jj]],
                              sem_s.at[jj % NBUF]).wait()
    plsc.subcore_barrier()
    pltpu.sync_copy(acc_sh.at[pl.ds(base, ROWS_PER_TILE)],
                    out_hbm.at[c, pl.ds(base, ROWS_PER_TILE)])


GRID = 4
BLK = N_PAD // GRID


def _blk(shape2):
    return pl.BlockSpec((BLK,) + shape2, lambda i: (i,) + (0,) * len(shape2))


def _full(shape):
    return pl.BlockSpec(shape, lambda i: (0,) * len(shape))


def _tc1a_body(x_ref, w_ref, h_ref):
    h_ref[...] = jnp.dot(x_ref[...], w_ref[...],
                         preferred_element_type=jnp.float32)


def _tc1b_body(deg_ref, h_ref, p_ref, dinv_ref):
    d = deg_ref[:, 0:1] + deg_ref[:, 1:2] + 1.0
    dinv = lax.rsqrt(d)
    p_ref[...] = h_ref[...] * dinv
    dinv_ref[...] = dinv


def _tc2_body(acc_ref, p_ref, dinv_ref, b_ref, w_ref, out_ref):
    agg = acc_ref[0] + acc_ref[1] + p_ref[...]
    h = jnp.maximum(agg * dinv_ref[...] + b_ref[...], 0.0)
    out_ref[...] = jnp.dot(h, w_ref[...],
                           preferred_element_type=jnp.float32) * dinv_ref[...]


def _tc3_body(acc_ref, p_ref, dinv_ref, b_ref, w_ref, bh_ref, mask_ref, out_ref):
    agg = acc_ref[0] + acc_ref[1] + p_ref[...]
    h = jnp.maximum(agg * dinv_ref[...] + b_ref[...], 0.0)
    logits = jnp.dot(h, w_ref[...], preferred_element_type=jnp.float32) + bh_ref[...]
    neg = jnp.float32(jnp.finfo(jnp.float32).min)
    out_ref[...] = jnp.where(mask_ref[...] > 0, logits, neg)


def kernel(x, edge_index, action_mask, W1, b1, W2, b2, Wh, bh):
    f32 = jnp.float32
    edge3 = edge_index.reshape(NC_EDGE, NQ, CHUNK)
    xpad = jnp.pad(x, ((0, N_PAD - N), (0, 0)))
    maskp = jnp.pad(action_mask, ((0, N_PAD - N), (0, 0))).astype(jnp.int32)
    zeros1 = jnp.zeros((ROWS_PER_TILE,), f32)
    zeros2 = jnp.zeros((ROWS_PER_TILE, HID), f32)
    ones_c = jnp.ones((CHUNK,), f32)
    b1r = b1.reshape(1, HID)
    b2r = b2.reshape(1, HID)
    bhr = bh.reshape(1, NUM_COLORS)

    deg = _sc_degree(edge3, ones_c, zeros1)
    degT = deg.T
    acc_spec = pl.BlockSpec((NC, BLK, HID), lambda i: (0, i, 0))
    h1 = pl.pallas_call(
        _tc1a_body,
        grid=(GRID,),
        in_specs=[_blk((IN_DIM,)), _full((IN_DIM, HID))],
        out_specs=_blk((HID,)),
        out_shape=jax.ShapeDtypeStruct((N_PAD, HID), f32),
    )(xpad, W1)

    p1s, dinv = pl.pallas_call(
        _tc1b_body,
        grid=(GRID,),
        in_specs=[_blk((2,)), _blk((HID,))],
        out_specs=[_blk((HID,)), _blk((1,))],
        out_shape=[jax.ShapeDtypeStruct((N_PAD, HID), f32),
                   jax.ShapeDtypeStruct((N_PAD, 1), f32)],
    )(degT, h1)
    acc1 = _sc_aggregate(edge3, zeros2, p1s)

    p2s = pl.pallas_call(
        _tc2_body,
        grid=(GRID,),
        in_specs=[acc_spec, _blk((HID,)), _blk((1,)), _full((1, HID)),
                  _full((HID, HID))],
        out_specs=_blk((HID,)),
        out_shape=jax.ShapeDtypeStruct((N_PAD, HID), f32),
    )(acc1, p1s, dinv, b1r, W2)
    acc2 = _sc_aggregate(edge3, zeros2, p2s)

    outp = pl.pallas_call(
        _tc3_body,
        grid=(GRID,),
        in_specs=[acc_spec, _blk((HID,)), _blk((1,)), _full((1, HID)),
                  _full((HID, NUM_COLORS)), _full((1, NUM_COLORS)),
                  _blk((NUM_COLORS,))],
        out_specs=_blk((NUM_COLORS,)),
        out_shape=jax.ShapeDtypeStruct((N_PAD, NUM_COLORS), f32),
    )(acc2, p2s, dinv, b2r, Wh, bhr, maskp)
    return outp[:N].reshape(-1)


# speedup vs baseline: 1.1312x; 1.0447x over previous
import functools

import jax
import jax.numpy as jnp
import numpy as _np
from jax import lax
from jax.experimental import pallas as pl
from jax.experimental.pallas import tpu as pltpu
from jax.experimental.pallas import tpu_sc as plsc

N = 10000
E = 320000
IN_DIM = 128
HID = 64
NUM_COLORS = 16

NC = 2
NS = 16
CHUNK = 128
NQ = E // CHUNK
NW = NC * NS
NC_EDGE = 2
N_CHUNKS = 79
N_PAD = 10240
ROWS_PER_TILE = N_PAD // NS

_mesh = plsc.VectorSubcoreMesh(
    core_axis_name="c", subcore_axis_name="s", num_cores=NC, num_subcores=NS)
_sc_params = pltpu.CompilerParams(use_tc_tiling_on_sc=False)


@functools.partial(
    pl.kernel,
    out_type=jax.ShapeDtypeStruct((NC, N_PAD), jnp.float32),
    mesh=_mesh,
    scratch_types=[
        pltpu.VMEM((N_CHUNKS, CHUNK), jnp.int32),
        pltpu.VMEM((CHUNK,), jnp.float32),
        pltpu.VMEM_SHARED((N_PAD,), jnp.float32),
        pltpu.SemaphoreType.DMA,
    ],
    compiler_params=_sc_params,
)
def _sc_degree(edges_hbm, ones_hbm, zeros_hbm, out_hbm, dst_v, ones_v, deg_sh, sem):
    c = lax.axis_index("c")
    s = lax.axis_index("s")
    wid = c * NS + s
    q_lo = (wid * NQ) // NW
    nq = ((wid + 1) * NQ) // NW - q_lo
    base = s * ROWS_PER_TILE
    pltpu.sync_copy(zeros_hbm, deg_sh.at[pl.ds(base, ROWS_PER_TILE)])
    pltpu.sync_copy(ones_hbm, ones_v)
    pltpu.sync_copy(edges_hbm.at[1, pl.ds(q_lo, N_CHUNKS)], dst_v)
    plsc.subcore_barrier()

    def body(j, carry):
        pltpu.sync_copy(ones_v, deg_sh.at[dst_v.at[j]], add=True)
        return carry

    lax.fori_loop(0, nq, body, 0)
    plsc.subcore_barrier()
    pltpu.sync_copy(deg_sh.at[pl.ds(base, ROWS_PER_TILE)],
                    out_hbm.at[c, pl.ds(base, ROWS_PER_TILE)])


@functools.partial(
    pl.kernel,
    out_type=jax.ShapeDtypeStruct((NC, N_PAD, HID), jnp.float32),
    mesh=_mesh,
    scratch_types=[
        pltpu.VMEM((N_CHUNKS, CHUNK), jnp.int32),
        pltpu.VMEM((N_CHUNKS, CHUNK), jnp.int32),
        pltpu.VMEM((6, CHUNK, HID), jnp.float32),
        pltpu.VMEM_SHARED((N_PAD, HID), jnp.float32),
        pltpu.SemaphoreType.DMA((6,)),
        pltpu.SemaphoreType.DMA((6,)),
    ],
    compiler_params=_sc_params,
)
def _sc_aggregate(edges_hbm, zeros_hbm, tab_hbm, out_hbm,
                  src_v, dst_v, rows_v, acc_sh, sem_g, sem_s):
    c = lax.axis_index("c")
    s = lax.axis_index("s")
    wid = c * NS + s
    q_lo = (wid * NQ) // NW
    nq = ((wid + 1) * NQ) // NW - q_lo
    base = s * ROWS_PER_TILE
    pltpu.sync_copy(zeros_hbm, acc_sh.at[pl.ds(base, ROWS_PER_TILE)])
    pltpu.sync_copy(edges_hbm.at[0, pl.ds(q_lo, N_CHUNKS)], src_v)
    pltpu.sync_copy(edges_hbm.at[1, pl.ds(q_lo, N_CHUNKS)], dst_v)
    plsc.subcore_barrier()

    NBUF = 6
    for j0 in range(NBUF - 1):
        pltpu.async_copy(tab_hbm.at[src_v.at[j0]], rows_v.at[j0], sem_g.at[j0])

    def body(j, carry):
        b = lax.rem(j, NBUF)
        pltpu.make_async_copy(tab_hbm.at[src_v.at[j]], rows_v.at[b],
                              sem_g.at[b]).wait()
        pltpu.async_copy(rows_v.at[b], acc_sh.at[dst_v.at[j]], sem_s.at[b],
                         add=True)
        k = j + NBUF - 1

        @pl.when(k < nq)
        def _():
            bk = lax.rem(k, NBUF)

            @pl.when(j >= 1)
            def _():
                pltpu.make_async_copy(rows_v.at[bk], acc_sh.at[dst_v.at[j - 1]],
                                      sem_s.at[bk]).wait()

            pltpu.async_copy(tab_hbm.at[src_v.at[k]], rows_v.at[bk], sem_g.at[bk])

        return carry

    lax.fori_loop(0, nq, body, 0)
    for i in range(NBUF):
        jj = nq - NBUF + i
        pltpu.make_async_copy(rows_v.at[jj % NBUF], acc_sh.at[dst_v.at[jj]],
                              sem_s.at[jj % NBUF]).wait()
    plsc.subcore_barrier()
    pltpu.sync_copy(acc_sh.at[pl.ds(base, ROWS_PER_TILE)],
                    out_hbm.at[c, pl.ds(base, ROWS_PER_TILE)])


GRID = 2
BLK = N_PAD // GRID


def _blk(shape2):
    return pl.BlockSpec((BLK,) + shape2, lambda i: (i,) + (0,) * len(shape2))


def _full(shape):
    return pl.BlockSpec(shape, lambda i: (0,) * len(shape))


def _tc1a_body(x_ref, w_ref, h_ref):
    h_ref[...] = jnp.dot(x_ref[...], w_ref[...],
                         preferred_element_type=jnp.float32)


def _tc1b_body(deg_ref, h_ref, p_ref, dinv_ref):
    d = deg_ref[:, 0:1] + deg_ref[:, 1:2] + 1.0
    dinv = lax.rsqrt(d)
    p_ref[...] = h_ref[...] * dinv
    dinv_ref[...] = dinv


def _tc2_body(acc_ref, p_ref, dinv_ref, b_ref, w_ref, out_ref):
    agg = acc_ref[0] + acc_ref[1] + p_ref[...]
    h = jnp.maximum(agg * dinv_ref[...] + b_ref[...], 0.0)
    out_ref[...] = jnp.dot(h, w_ref[...],
                           preferred_element_type=jnp.float32) * dinv_ref[...]


def _tc3_body(acc_ref, p_ref, dinv_ref, b_ref, w_ref, bh_ref, mask_ref, out_ref):
    agg = acc_ref[0] + acc_ref[1] + p_ref[...]
    h = jnp.maximum(agg * dinv_ref[...] + b_ref[...], 0.0)
    logits = jnp.dot(h, w_ref[...], preferred_element_type=jnp.float32) + bh_ref[...]
    neg = jnp.float32(jnp.finfo(jnp.float32).min)
    out_ref[...] = jnp.where(mask_ref[...] > 0, logits, neg)


def kernel(x, edge_index, action_mask, W1, b1, W2, b2, Wh, bh):
    f32 = jnp.float32
    edge3 = edge_index.reshape(NC_EDGE, NQ, CHUNK)
    xpad = jnp.pad(x, ((0, N_PAD - N), (0, 0)))
    maskp = jnp.pad(action_mask, ((0, N_PAD - N), (0, 0))).astype(jnp.int32)
    zeros1 = _np.zeros((ROWS_PER_TILE,), _np.float32)
    zeros2 = _np.zeros((ROWS_PER_TILE, HID), _np.float32)
    ones_c = _np.ones((CHUNK,), _np.float32)
    b1r = b1.reshape(1, HID)
    b2r = b2.reshape(1, HID)
    bhr = bh.reshape(1, NUM_COLORS)

    deg = _sc_degree(edge3, ones_c, zeros1)
    degT = deg.T
    acc_spec = pl.BlockSpec((NC, BLK, HID), lambda i: (0, i, 0))
    h1 = pl.pallas_call(
        _tc1a_body,
        grid=(GRID,),
        in_specs=[_blk((IN_DIM,)), _full((IN_DIM, HID))],
        out_specs=_blk((HID,)),
        out_shape=jax.ShapeDtypeStruct((N_PAD, HID), f32),
    )(xpad, W1)

    p1s, dinv = pl.pallas_call(
        _tc1b_body,
        grid=(GRID,),
        in_specs=[_blk((2,)), _blk((HID,))],
        out_specs=[_blk((HID,)), _blk((1,))],
        out_shape=[jax.ShapeDtypeStruct((N_PAD, HID), f32),
                   jax.ShapeDtypeStruct((N_PAD, 1), f32)],
    )(degT, h1)
    acc1 = _sc_aggregate(edge3, zeros2, p1s)

    p2s = pl.pallas_call(
        _tc2_body,
        grid=(GRID,),
        in_specs=[acc_spec, _blk((HID,)), _blk((1,)), _full((1, HID)),
                  _full((HID, HID))],
        out_specs=_blk((HID,)),
        out_shape=jax.ShapeDtypeStruct((N_PAD, HID), f32),
    )(acc1, p1s, dinv, b1r, W2)
    acc2 = _sc_aggregate(edge3, zeros2, p2s)

    outp = pl.pallas_call(
        _tc3_body,
        grid=(GRID,),
        in_specs=[acc_spec, _blk((HID,)), _blk((1,)), _full((1, HID)),
                  _full((HID, NUM_COLORS)), _full((1, NUM_COLORS)),
                  _blk((NUM_COLORS,))],
        out_specs=_blk((NUM_COLORS,)),
        out_shape=jax.ShapeDtypeStruct((N_PAD, NUM_COLORS), f32),
    )(acc2, p2s, dinv, b2r, Wh, bhr, maskp)
    return outp[:N].reshape(-1)
